# baseline (speedup 1.0000x reference)
import jax
import jax.numpy as jnp
from jax import lax
from jax.experimental import pallas as pl
from jax.experimental.pallas import tpu as pltpu
from jax.experimental.pallas import tpu_sc as plsc

B = 4096
F = 26
D = 32
NFEAT = 100000
NW = 32
RPW = B // NW
IPW = RPW * F
CROWS = 16
CIDX = CROWS * F
NCHUNK = RPW // CROWS
NBUF = 3

_DNUMS = lax.GatherDimensionNumbers(
    offset_dims=(), collapsed_slice_dims=(0,), start_index_map=(0,))


def _vec_helpers():
    lane = lax.iota(jnp.int32, 16)
    mask10 = jnp.where(lane < 10, 1.0, 0.0).astype(jnp.float32)
    perms = [jnp.bitwise_xor(lane, sh) for sh in (1, 2, 4, 8)]

    def allsum(t):
        for p in perms:
            t = t + lax.gather(t, p[:, None], _DNUMS, slice_sizes=(1,),
                               mode=lax.GatherScatterMode.PROMISE_IN_BOUNDS)
        return t

    return lane, mask10, allsum


def _wid():
    nc = plsc.get_sparse_core_info().num_cores
    return lax.axis_index("s") * nc + lax.axis_index("c")


def _first_body(fi_hbm, fv_hbm, fw_hbm, bias_hbm, y1_hbm,
                idx_v, fv_v, fw_tab, out_v, bias_v, sem_tab):
    wid = _wid()
    htab = pltpu.async_copy(fw_hbm, fw_tab, sem_tab)
    pltpu.sync_copy(fi_hbm.at[pl.ds(wid * IPW, IPW)], idx_v.at[pl.ds(0, IPW)])
    pltpu.sync_copy(fv_hbm.at[pl.ds(wid * IPW, IPW)], fv_v.at[pl.ds(0, IPW)])
    pltpu.sync_copy(bias_hbm, bias_v)
    htab.wait()

    lane, mask10, allsum = _vec_helpers()
    z = jnp.zeros((16,), jnp.float32)

    def grp_body(g, carry):
        def row_body(j, acc):
            k0 = (g * 16 + j) * F
            v0 = fv_v[pl.ds(k0, 16)]
            v1 = fv_v[pl.ds(k0 + 16, 16)] * mask10
            i0 = idx_v[pl.ds(k0, 16)]
            i1 = jnp.where(lane < 10, idx_v[pl.ds(k0 + 16, 16)], 0)
            t = plsc.load_gather(fw_tab, [i0]) * v0 \
                + plsc.load_gather(fw_tab, [i1]) * v1
            return jnp.where(lane == j, allsum(t), acc)

        acc = lax.fori_loop(0, 16, row_body, z)
        out_v[pl.ds(g * 16, 16)] = acc + bias_v[...]
        return carry

    lax.fori_loop(0, RPW // 16, grp_body, 0)
    pltpu.sync_copy(out_v, y1_hbm.at[pl.ds(wid * RPW, RPW)])


def _secd_body(fi_hbm, fv_hbm, emb_hbm, y1_hbm, out_hbm,
               idx_v, fv_v, rows_v, y1_v, out_v,
               sem_a, sem_b, sem_c):
    wid = _wid()
    pltpu.sync_copy(fi_hbm.at[pl.ds(wid * IPW, IPW)], idx_v.at[pl.ds(0, IPW)])

    sems = [sem_a, sem_b, sem_c]

    def issue(c):
        return pltpu.async_copy(
            emb_hbm.at[idx_v.at[pl.ds(c * CIDX, CIDX)]],
            rows_v.at[pl.ds((c % NBUF) * CIDX, CIDX)],
            sems[c % NBUF])

    handles = [issue(0), issue(1), None]
    pltpu.sync_copy(fv_hbm.at[pl.ds(wid * IPW, IPW)], fv_v.at[pl.ds(0, IPW)])
    pltpu.sync_copy(y1_hbm.at[pl.ds(wid * RPW, RPW)], y1_v)

    lane, mask10, allsum = _vec_helpers()
    z = jnp.zeros((16,), jnp.float32)

    for c in range(NCHUNK):
        if c + 2 < NCHUNK:
            handles[(c + 2) % NBUF] = issue(c + 2)
        handles[c % NBUF].wait()
        rbase = (c % NBUF) * CIDX

        def row_body(j, acc, c=c, rbase=rbase):
            k0 = (c * CROWS + j) * F
            v0 = fv_v[pl.ds(k0, 16)]
            v1 = fv_v[pl.ds(k0 + 16, 16)] * mask10
            s0, s1, q0, q1 = z, z, z, z
            for f in range(F):
                fvs = v0[f] if f < 16 else v1[f - 16]
                r = rbase + j * F + f
                e0 = rows_v[r, pl.ds(0, 16)] * fvs
                e1 = rows_v[r, pl.ds(16, 16)] * fvs
                s0 = s0 + e0
                s1 = s1 + e1
                q0 = q0 + e0 * e0
                q1 = q1 + e1 * e1
            t = ((s0 * s0 - q0) + (s1 * s1 - q1)) * 0.5
            return jnp.where(lane == j, allsum(t), acc)

        acc = lax.fori_loop(0, CROWS, row_body, z)
        out_v[pl.ds(c * CROWS, 16)] = acc + y1_v[pl.ds(c * CROWS, 16)]

    pltpu.sync_copy(out_v, out_hbm.at[pl.ds(wid * RPW, RPW)])


@jax.jit
def _fm_call(fi2, fv2, fw1, emb1, bias16):
    emb = emb1.reshape(NFEAT, D)
    mesh = plsc.VectorSubcoreMesh(core_axis_name="c", subcore_axis_name="s")
    params = pltpu.CompilerParams(
        use_tc_tiling_on_sc=False, needs_layout_passes=False)

    first = pl.kernel(
        _first_body,
        out_type=jax.ShapeDtypeStruct((B,), jnp.float32),
        mesh=mesh,
        compiler_params=params,
        scratch_types=[
            pltpu.VMEM((IPW + 16,), jnp.int32),
            pltpu.VMEM((IPW + 16,), jnp.float32),
            pltpu.VMEM((NFEAT,), jnp.float32),
            pltpu.VMEM((RPW,), jnp.float32),
            pltpu.VMEM((16,), jnp.float32),
            pltpu.SemaphoreType.DMA,
        ],
    )
    y1 = first(fi2, fv2, fw1, bias16)

    secd = pl.kernel(
        _secd_body,
        out_type=jax.ShapeDtypeStruct((B,), jnp.float32),
        mesh=mesh,
        compiler_params=params,
        scratch_types=[
            pltpu.VMEM((IPW + 16,), jnp.int32),
            pltpu.VMEM((IPW + 16,), jnp.float32),
            pltpu.VMEM((NBUF * CIDX, D), jnp.float32),
            pltpu.VMEM((RPW,), jnp.float32),
            pltpu.VMEM((RPW,), jnp.float32),
            pltpu.SemaphoreType.DMA,
            pltpu.SemaphoreType.DMA,
            pltpu.SemaphoreType.DMA,
        ],
    )
    return secd(fi2, fv2, emb, y1)


def kernel(feat_index, feat_value, first_weights, feat_embeddings, bias):
    emb1 = feat_embeddings.reshape(NFEAT * D)
    fi2 = feat_index.astype(jnp.int32).reshape(NW * IPW)
    fv2 = feat_value.reshape(NW * IPW)
    fw1 = first_weights.reshape(NFEAT)
    bias16 = jnp.broadcast_to(bias, (16,))
    out = _fm_call(fi2, fv2, fw1, emb1, bias16)
    return out.reshape(B, 1)


# speedup vs baseline: 1.0230x; 1.0230x over previous
import jax
import jax.numpy as jnp
from jax import lax
from jax.experimental import pallas as pl
from jax.experimental.pallas import tpu as pltpu
from jax.experimental.pallas import tpu_sc as plsc

B = 4096
F = 26
D = 32
NFEAT = 100000
NW = 32
RPW = B // NW
IPW = RPW * F
CROWS = 32
CIDX = CROWS * F
NCHUNK = RPW // CROWS
NBUF = 3

_DNUMS = lax.GatherDimensionNumbers(
    offset_dims=(), collapsed_slice_dims=(0,), start_index_map=(0,))


def _vec_helpers():
    lane = lax.iota(jnp.int32, 16)
    mask10 = jnp.where(lane < 10, 1.0, 0.0).astype(jnp.float32)
    perms = [jnp.bitwise_xor(lane, sh) for sh in (1, 2, 4, 8)]

    def allsum(t):
        for p in perms:
            t = t + lax.gather(t, p[:, None], _DNUMS, slice_sizes=(1,),
                               mode=lax.GatherScatterMode.PROMISE_IN_BOUNDS)
        return t

    return lane, mask10, allsum


def _wid():
    nc = plsc.get_sparse_core_info().num_cores
    return lax.axis_index("s") * nc + lax.axis_index("c")


def _first_body(fi_hbm, fv_hbm, fw_hbm, bias_hbm, y1_hbm,
                idx_v, fv_v, fw_tab, out_v, bias_v, sem_tab):
    wid = _wid()
    htab = pltpu.async_copy(fw_hbm, fw_tab, sem_tab)
    pltpu.sync_copy(fi_hbm.at[pl.ds(wid * IPW, IPW)], idx_v.at[pl.ds(0, IPW)])
    pltpu.sync_copy(fv_hbm.at[pl.ds(wid * IPW, IPW)], fv_v.at[pl.ds(0, IPW)])
    pltpu.sync_copy(bias_hbm, bias_v)
    htab.wait()

    lane, mask10, allsum = _vec_helpers()
    z = jnp.zeros((16,), jnp.float32)

    def grp_body(g, carry):
        def row_body(j, acc):
            k0 = (g * 16 + j) * F
            v0 = fv_v[pl.ds(k0, 16)]
            v1 = fv_v[pl.ds(k0 + 16, 16)] * mask10
            i0 = idx_v[pl.ds(k0, 16)]
            i1 = jnp.where(lane < 10, idx_v[pl.ds(k0 + 16, 16)], 0)
            t = plsc.load_gather(fw_tab, [i0]) * v0 \
                + plsc.load_gather(fw_tab, [i1]) * v1
            return jnp.where(lane == j, allsum(t), acc)

        acc = lax.fori_loop(0, 16, row_body, z)
        out_v[pl.ds(g * 16, 16)] = acc + bias_v[...]
        return carry

    lax.fori_loop(0, RPW // 16, grp_body, 0)
    pltpu.sync_copy(out_v, y1_hbm.at[pl.ds(wid * RPW, RPW)])


def _secd_body(fi_hbm, fv_hbm, emb_hbm, y1_hbm, out_hbm,
               idx_v, fv_v, rows_v, y1_v, out_v,
               sem_a, sem_b, sem_c):
    wid = _wid()
    pltpu.sync_copy(fi_hbm.at[pl.ds(wid * IPW, IPW)], idx_v.at[pl.ds(0, IPW)])

    sems = [sem_a, sem_b, sem_c]

    def issue(c):
        return pltpu.async_copy(
            emb_hbm.at[idx_v.at[pl.ds(c * CIDX, CIDX)]],
            rows_v.at[pl.ds((c % NBUF) * CIDX, CIDX)],
            sems[c % NBUF])

    handles = [issue(0), issue(1), None]
    pltpu.sync_copy(fv_hbm.at[pl.ds(wid * IPW, IPW)], fv_v.at[pl.ds(0, IPW)])
    pltpu.sync_copy(y1_hbm.at[pl.ds(wid * RPW, RPW)], y1_v)

    lane, mask10, allsum = _vec_helpers()
    z = jnp.zeros((16,), jnp.float32)

    for c in range(NCHUNK):
        if c + 2 < NCHUNK:
            handles[(c + 2) % NBUF] = issue(c + 2)
        handles[c % NBUF].wait()
        rbase = (c % NBUF) * CIDX

        def row_body(j, accs, c=c, rbase=rbase):
            acc0, acc1 = accs
            k0 = (c * CROWS + j) * F
            v0 = fv_v[pl.ds(k0, 16)]
            v1 = fv_v[pl.ds(k0 + 16, 16)] * mask10
            s0, s1, q0, q1 = z, z, z, z
            for f in range(F):
                fvs = v0[f] if f < 16 else v1[f - 16]
                r = rbase + j * F + f
                e0 = rows_v[r, pl.ds(0, 16)] * fvs
                e1 = rows_v[r, pl.ds(16, 16)] * fvs
                s0 = s0 + e0
                s1 = s1 + e1
                q0 = q0 + e0 * e0
                q1 = q1 + e1 * e1
            t = ((s0 * s0 - q0) + (s1 * s1 - q1)) * 0.5
            y = allsum(t)
            return (jnp.where(lane == j, y, acc0),
                    jnp.where(lane == j - 16, y, acc1))

        acc0, acc1 = lax.fori_loop(0, CROWS, row_body, (z, z))
        out_v[pl.ds(c * CROWS, 16)] = acc0 + y1_v[pl.ds(c * CROWS, 16)]
        out_v[pl.ds(c * CROWS + 16, 16)] = (
            acc1 + y1_v[pl.ds(c * CROWS + 16, 16)])

    pltpu.sync_copy(out_v, out_hbm.at[pl.ds(wid * RPW, RPW)])


@jax.jit
def _fm_call(fi2, fv2, fw1, emb1, bias16):
    emb = emb1.reshape(NFEAT, D)
    mesh = plsc.VectorSubcoreMesh(core_axis_name="c", subcore_axis_name="s")
    params = pltpu.CompilerParams(
        use_tc_tiling_on_sc=False, needs_layout_passes=False)

    first = pl.kernel(
        _first_body,
        out_type=jax.ShapeDtypeStruct((B,), jnp.float32),
        mesh=mesh,
        compiler_params=params,
        scratch_types=[
            pltpu.VMEM((IPW + 16,), jnp.int32),
            pltpu.VMEM((IPW + 16,), jnp.float32),
            pltpu.VMEM((NFEAT,), jnp.float32),
            pltpu.VMEM((RPW,), jnp.float32),
            pltpu.VMEM((16,), jnp.float32),
            pltpu.SemaphoreType.DMA,
        ],
    )
    y1 = first(fi2, fv2, fw1, bias16)

    secd = pl.kernel(
        _secd_body,
        out_type=jax.ShapeDtypeStruct((B,), jnp.float32),
        mesh=mesh,
        compiler_params=params,
        scratch_types=[
            pltpu.VMEM((IPW + 16,), jnp.int32),
            pltpu.VMEM((IPW + 16,), jnp.float32),
            pltpu.VMEM((NBUF * CIDX, D), jnp.float32),
            pltpu.VMEM((RPW,), jnp.float32),
            pltpu.VMEM((RPW,), jnp.float32),
            pltpu.SemaphoreType.DMA,
            pltpu.SemaphoreType.DMA,
            pltpu.SemaphoreType.DMA,
        ],
    )
    return secd(fi2, fv2, emb, y1)


def kernel(feat_index, feat_value, first_weights, feat_embeddings, bias):
    emb1 = feat_embeddings.reshape(NFEAT * D)
    fi2 = feat_index.astype(jnp.int32).reshape(NW * IPW)
    fv2 = feat_value.reshape(NW * IPW)
    fw1 = first_weights.reshape(NFEAT)
    bias16 = jnp.broadcast_to(bias, (16,))
    out = _fm_call(fi2, fv2, fw1, emb1, bias16)
    return out.reshape(B, 1)
